# baseline (speedup 1.0000x reference)
import jax
import jax.numpy as jnp
from jax.experimental import pallas as pl
from jax.experimental.pallas import tpu as pltpu

_N, _L, _C, _H, _G = 50000, 9, 256, 512, 256
_BLK = 2000
_NBLK = _N // _BLK


def _mlp_energy_body(b3_ref, x_ref, bidx_ref, w1h_ref, w1l_ref, b1_ref,
                     w2h_ref, w2l_ref, b2_ref, w3_ref, out_ref):
    i = pl.program_id(0)
    x = x_ref[:, :]
    h = (jnp.dot(x, w1h_ref[:], preferred_element_type=jnp.float32)
         + jnp.dot(x, w1l_ref[:], preferred_element_type=jnp.float32)
         + b1_ref[:])
    h = h * jax.nn.sigmoid(h)
    hb = h.astype(jnp.bfloat16)
    h = (jnp.dot(hb, w2h_ref[:], preferred_element_type=jnp.float32)
         + jnp.dot(hb, w2l_ref[:], preferred_element_type=jnp.float32)
         + b2_ref[:])
    h = h * jax.nn.sigmoid(h)
    e = jnp.sum(h * w3_ref[:], axis=1) + b3_ref[0]
    idx = bidx_ref[0, 0, :]
    onehot = (idx[:, None] == jax.lax.broadcasted_iota(
        jnp.int32, (_BLK, _G), 1)).astype(jnp.float32)
    part = jnp.dot(e[None, :], onehot, preferred_element_type=jnp.float32)

    @pl.when(i == 0)
    def _():
        out_ref[:] = jnp.zeros_like(out_ref)

    out_ref[:] += part


def _hilo(W):
    hi = W.astype(jnp.bfloat16)
    lo = (W - hi.astype(jnp.float32)).astype(jnp.bfloat16)
    return hi, lo


def kernel(node_embedding, batch, natoms, W1, b1, W2, b2, W3, b3):
    x_bf = node_embedding[:, 0, :].astype(jnp.bfloat16)
    w1h, w1l = _hilo(W1)
    w2h, w2l = _hilo(W2)
    bidx = batch.reshape(_NBLK, 1, _BLK)
    out = pl.pallas_call(
        _mlp_energy_body,
        grid=(_NBLK,),
        in_specs=[
            pl.BlockSpec(memory_space=pltpu.SMEM),
            pl.BlockSpec((_BLK, _C), lambda i: (i, 0)),
            pl.BlockSpec((1, 1, _BLK), lambda i: (i, 0, 0)),
            pl.BlockSpec((_C, _H), lambda i: (0, 0)),
            pl.BlockSpec((_C, _H), lambda i: (0, 0)),
            pl.BlockSpec((1, _H), lambda i: (0, 0)),
            pl.BlockSpec((_H, _H), lambda i: (0, 0)),
            pl.BlockSpec((_H, _H), lambda i: (0, 0)),
            pl.BlockSpec((1, _H), lambda i: (0, 0)),
            pl.BlockSpec((1, _H), lambda i: (0, 0)),
        ],
        out_specs=pl.BlockSpec((1, _G), lambda i: (0, 0)),
        out_shape=jax.ShapeDtypeStruct((1, _G), jnp.float32),
    )(b3, x_bf, bidx, w1h, w1l, b1.reshape(1, _H),
      w2h, w2l, b2.reshape(1, _H), W3.reshape(1, _H))
    return out[0]


# speedup vs baseline: 1.0424x; 1.0424x over previous
import jax
import jax.numpy as jnp
from jax.experimental import pallas as pl
from jax.experimental.pallas import tpu as pltpu

_N, _L, _C, _H, _G = 50000, 9, 256, 512, 256
_BLK = 5000
_NBLK = _N // _BLK


def _mlp_energy_body(b3_ref, x_ref, bidx_ref, w1h_ref, w1l_ref, b1_ref,
                     w2h_ref, w2l_ref, b2_ref, w3_ref, out_ref):
    i = pl.program_id(0)
    x = x_ref[:, :]
    h = (jnp.dot(x, w1h_ref[:], preferred_element_type=jnp.float32)
         + jnp.dot(x, w1l_ref[:], preferred_element_type=jnp.float32)
         + b1_ref[:])
    h = h * jax.nn.sigmoid(h)
    hb = h.astype(jnp.bfloat16)
    h = (jnp.dot(hb, w2h_ref[:], preferred_element_type=jnp.float32)
         + jnp.dot(hb, w2l_ref[:], preferred_element_type=jnp.float32)
         + b2_ref[:])
    h = h * jax.nn.sigmoid(h)
    e = jnp.sum(h * w3_ref[:], axis=1) + b3_ref[0]
    idx = bidx_ref[0, 0, :]
    onehot = (idx[:, None] == jax.lax.broadcasted_iota(
        jnp.int32, (_BLK, _G), 1)).astype(jnp.float32)
    part = jnp.dot(e[None, :], onehot, preferred_element_type=jnp.float32)

    @pl.when(i == 0)
    def _():
        out_ref[:] = jnp.zeros_like(out_ref)

    out_ref[:] += part


def _hilo(W):
    hi = W.astype(jnp.bfloat16)
    lo = (W - hi.astype(jnp.float32)).astype(jnp.bfloat16)
    return hi, lo


def kernel(node_embedding, batch, natoms, W1, b1, W2, b2, W3, b3):
    x_bf = node_embedding[:, 0, :].astype(jnp.bfloat16)
    w1h, w1l = _hilo(W1)
    w2h, w2l = _hilo(W2)
    bidx = batch.reshape(_NBLK, 1, _BLK)
    out = pl.pallas_call(
        _mlp_energy_body,
        grid=(_NBLK,),
        in_specs=[
            pl.BlockSpec(memory_space=pltpu.SMEM),
            pl.BlockSpec((_BLK, _C), lambda i: (i, 0)),
            pl.BlockSpec((1, 1, _BLK), lambda i: (i, 0, 0)),
            pl.BlockSpec((_C, _H), lambda i: (0, 0)),
            pl.BlockSpec((_C, _H), lambda i: (0, 0)),
            pl.BlockSpec((1, _H), lambda i: (0, 0)),
            pl.BlockSpec((_H, _H), lambda i: (0, 0)),
            pl.BlockSpec((_H, _H), lambda i: (0, 0)),
            pl.BlockSpec((1, _H), lambda i: (0, 0)),
            pl.BlockSpec((1, _H), lambda i: (0, 0)),
        ],
        out_specs=pl.BlockSpec((1, _G), lambda i: (0, 0)),
        out_shape=jax.ShapeDtypeStruct((1, _G), jnp.float32),
    )(b3, x_bf, bidx, w1h, w1l, b1.reshape(1, _H),
      w2h, w2l, b2.reshape(1, _H), W3.reshape(1, _H))
    return out[0]
